# baseline (speedup 1.0000x reference)
import functools

import jax
import jax.numpy as jnp
from jax import lax
from jax.experimental import pallas as pl
from jax.experimental.pallas import tpu as pltpu
from jax.experimental.pallas import tpu_sc as plsc

NC = 2
NS = 16
LANES = 16

CHUNK = 128
D = 64


def _sc_kernel_body(widx_hbm, pidx_hbm, word_hbm, pos_hbm, out_hbm,
                    widx_v, pidx_v, wrow_v, prow_v, sem_w, sem_p):
    wid = lax.axis_index("c") * NS + lax.axis_index("s")
    n_chunks_per_tile = widx_hbm.shape[0] // (NC * NS)

    def chunk_body(j, _):
        row = wid * n_chunks_per_tile + j
        pltpu.sync_copy(widx_hbm.at[row], widx_v)
        pltpu.sync_copy(pidx_hbm.at[row], pidx_v)
        cw = pltpu.async_copy(word_hbm.at[widx_v], wrow_v, sem_w)
        cp = pltpu.async_copy(pos_hbm.at[pidx_v], prow_v, sem_p)
        cw.wait()
        cp.wait()

        def add_body(r, _):
            for c in range(D // LANES):
                sl = pl.ds(c * LANES, LANES)
                prow_v[r, sl] = prow_v[r, sl] + wrow_v[r, sl]
            return 0

        lax.fori_loop(0, CHUNK, add_body, 0, unroll=False)
        pltpu.sync_copy(prow_v, out_hbm.at[pl.ds(row * CHUNK, CHUNK)])
        return 0

    lax.fori_loop(0, n_chunks_per_tile, chunk_body, 0, unroll=False)


def kernel(src_word, src_pos, word_emb, pos_emb):
    B, S = src_word.shape
    n = B * S
    n_rows = n // CHUNK
    widx = src_word.reshape(n_rows, CHUNK)
    pidx = src_pos.reshape(n_rows, CHUNK)

    mesh = plsc.VectorSubcoreMesh(core_axis_name="c", subcore_axis_name="s",
                                  num_cores=NC, num_subcores=NS)
    run = pl.kernel(
        _sc_kernel_body,
        out_type=jax.ShapeDtypeStruct((n, D), jnp.float32),
        mesh=mesh,
        compiler_params=pltpu.CompilerParams(use_tc_tiling_on_sc=False),
        scratch_types=[
            pltpu.VMEM((CHUNK,), jnp.int32),
            pltpu.VMEM((CHUNK,), jnp.int32),
            pltpu.VMEM((CHUNK, D), jnp.float32),
            pltpu.VMEM((CHUNK, D), jnp.float32),
            pltpu.SemaphoreType.DMA,
            pltpu.SemaphoreType.DMA,
        ],
    )
    out = run(widx, pidx, word_emb, pos_emb)
    return out.reshape(B, S, D)


# speedup vs baseline: 1.0296x; 1.0296x over previous
import jax
import jax.numpy as jnp
from jax import lax
from jax.experimental import pallas as pl
from jax.experimental.pallas import tpu as pltpu
from jax.experimental.pallas import tpu_sc as plsc

NC = 2
NS = 16
LANES = 16

CHUNK = 128
D = 64


def _sc_kernel_body(widx_hbm, pidx_hbm, word_hbm, pos_hbm, out_hbm,
                    widx_v, pidx_v, buf0, buf1,
                    semp0, semp1, semw0, semw1, sems0, sems1):
    wid = lax.axis_index("c") * NS + lax.axis_index("s")
    n_chunks = widx_hbm.shape[0] // (NC * NS)
    row0 = wid * n_chunks

    pltpu.sync_copy(widx_hbm.at[pl.ds(row0, n_chunks)], widx_v)
    pltpu.sync_copy(pidx_hbm.at[pl.ds(row0, n_chunks)], pidx_v)

    bufs = [buf0, buf1]
    semps = [semp0, semp1]
    semws = [semw0, semw1]
    semss = [sems0, sems1]
    cp = [None, None]
    cw = [None, None]
    cs = [None, None]

    cp[0] = pltpu.async_copy(pos_hbm.at[pidx_v.at[0]], bufs[0], semps[0])

    for j in range(n_chunks):
        b = j % 2
        nb = (j + 1) % 2
        cp[b].wait()
        cw[b] = pltpu.async_copy(word_hbm.at[widx_v.at[j]], bufs[b],
                                 semws[b], add=True)
        if j + 1 < n_chunks:
            if cs[nb] is not None:
                cs[nb].wait()
            cp[nb] = pltpu.async_copy(pos_hbm.at[pidx_v.at[j + 1]], bufs[nb],
                                      semps[nb])
        cw[b].wait()
        cs[b] = pltpu.async_copy(bufs[b],
                                 out_hbm.at[pl.ds((row0 + j) * CHUNK, CHUNK)],
                                 semss[b])

    for c in cs:
        if c is not None:
            c.wait()


def kernel(src_word, src_pos, word_emb, pos_emb):
    B, S = src_word.shape
    n = B * S
    n_rows = n // CHUNK
    widx = src_word.reshape(n_rows, CHUNK)
    pidx = src_pos.reshape(n_rows, CHUNK)

    mesh = plsc.VectorSubcoreMesh(core_axis_name="c", subcore_axis_name="s",
                                  num_cores=NC, num_subcores=NS)
    run = pl.kernel(
        _sc_kernel_body,
        out_type=jax.ShapeDtypeStruct((n, D), jnp.float32),
        mesh=mesh,
        compiler_params=pltpu.CompilerParams(use_tc_tiling_on_sc=False),
        scratch_types=[
            pltpu.VMEM((n_rows // (NC * NS), CHUNK), jnp.int32),
            pltpu.VMEM((n_rows // (NC * NS), CHUNK), jnp.int32),
            pltpu.VMEM((CHUNK, D), jnp.float32),
            pltpu.VMEM((CHUNK, D), jnp.float32),
            pltpu.SemaphoreType.DMA,
            pltpu.SemaphoreType.DMA,
            pltpu.SemaphoreType.DMA,
            pltpu.SemaphoreType.DMA,
            pltpu.SemaphoreType.DMA,
            pltpu.SemaphoreType.DMA,
        ],
    )
    out = run(widx, pidx, word_emb, pos_emb)
    return out.reshape(B, S, D)
